# Initial kernel scaffold; baseline (speedup 1.0000x reference)
import jax
import jax.numpy as jnp
from jax.experimental import pallas as pl


def kernel(x, edge_index, W0, att_src0, att_dst0, b0, W1, att_src1, att_dst1, b1, W2, att_src2, att_dst2, b2, ln0_g, ln0_b, ln1_g, ln1_b, ln2_g, ln2_b, wq, bq, wk, bk, wv, bv, wo, bo):
    raise NotImplementedError("write your pallas kernel here")


# speedup vs baseline: 27.9718x; 27.9718x over previous
import functools

import jax
import jax.numpy as jnp
from jax import lax
from jax.experimental import pallas as pl
from jax.experimental.pallas import tpu as pltpu

N = 4096
E = 131072
HID = 128
OUT = 128

BI = 512
BJ = 512
NI = N // BI
NJ = N // BJ

_pallas_call = pl.pallas_call

_DOT = functools.partial(lax.dot_general, preferred_element_type=jnp.float32)


def _dot_nt(a, b):
    return _DOT(a, b, dimension_numbers=(((1,), (1,)), ((), ())))


def _dot_nn(a, b):
    return _DOT(a, b, dimension_numbers=(((1,), (0,)), ((), ())))


def _build_count_matrix(edge_index):
    src = edge_index[0]
    dst = edge_index[1]
    flat = dst.astype(jnp.int32) * N + src.astype(jnp.int32)
    c = jnp.zeros((N * N,), jnp.float32).at[flat].add(1.0)
    loop = jnp.arange(N, dtype=jnp.int32)
    c = c.at[loop * N + loop].add(1.0)
    return c.reshape(N, N)


def _gat_pre_kernel(h_ref, w_ref, asrc_ref, adst_ref,
                    xp_ref, s_nh_ref, d_nh_ref, s_t_ref, *, heads):
    h = h_ref[...]
    xp = _dot_nt(h, w_ref[...])
    xp_ref[...] = xp
    for hh in range(heads):
        xph = xp[:, hh * 128:(hh + 1) * 128]
        a_s = jnp.sum(xph * asrc_ref[hh, :][None, :], axis=1, keepdims=True)
        a_d = jnp.sum(xph * adst_ref[hh, :][None, :], axis=1, keepdims=True)
        s_nh_ref[:, hh:hh + 1] = a_s
        d_nh_ref[:, hh:hh + 1] = a_d
        s_t_ref[hh:hh + 1, :] = _dot_nt(asrc_ref[hh:hh + 1, :], xph)


def _gat_pre(h, w, asrc, adst, heads):
    dout = heads * 128
    din = h.shape[1]
    return _pallas_call(
        functools.partial(_gat_pre_kernel, heads=heads),
        grid=(NI,),
        in_specs=[
            pl.BlockSpec((BI, din), lambda i: (i, 0)),
            pl.BlockSpec((dout, din), lambda i: (0, 0)),
            pl.BlockSpec((heads, 128), lambda i: (0, 0)),
            pl.BlockSpec((heads, 128), lambda i: (0, 0)),
        ],
        out_specs=[
            pl.BlockSpec((BI, dout), lambda i: (i, 0)),
            pl.BlockSpec((BI, heads), lambda i: (i, 0)),
            pl.BlockSpec((BI, heads), lambda i: (i, 0)),
            pl.BlockSpec((heads, BI), lambda i: (0, i)),
        ],
        out_shape=[
            jax.ShapeDtypeStruct((N, dout), jnp.float32),
            jax.ShapeDtypeStruct((N, heads), jnp.float32),
            jax.ShapeDtypeStruct((N, heads), jnp.float32),
            jax.ShapeDtypeStruct((heads, N), jnp.float32),
        ],
    )(h, w, asrc, adst)


def _gat_flash_kernel(c_ref, xp_ref, s_nh_ref, d_nh_ref, s_t_ref,
                      b_ref, g_ref, beta_ref, res_ref,
                      out_ref, acc_ref, den_ref,
                      *, heads, do_elu, do_res):
    j = pl.program_id(1)

    @pl.when(j == 0)
    def _init():
        acc_ref[...] = jnp.zeros_like(acc_ref)
        den_ref[...] = jnp.zeros_like(den_ref)

    cb = c_ref[...]
    s_full = s_nh_ref[...]
    s_max = jnp.max(s_full, axis=0, keepdims=True)
    d_blk = d_nh_ref[...]
    zsum = d_blk + s_max
    m_blk = jnp.where(zsum > 0, zsum, 0.2 * zsum)
    u = jnp.exp(d_blk - m_blk)
    p2 = jnp.exp(0.2 * d_blk - m_blk)
    s_t = s_t_ref[...]
    xp = xp_ref[...]

    for hh in range(heads):
        srow = s_t[hh:hh + 1, :]
        z = d_blk[:, hh:hh + 1] + srow
        pos = u[:, hh:hh + 1] * jnp.exp(srow)
        neg = p2[:, hh:hh + 1] * jnp.exp(0.2 * srow)
        p = cb * jnp.where(z > 0, pos, neg)
        acc_ref[:, hh * 128:(hh + 1) * 128] += _dot_nn(
            p, xp[:, hh * 128:(hh + 1) * 128])
        den_ref[:, hh:hh + 1] += jnp.sum(p, axis=1, keepdims=True)

    @pl.when(j == NJ - 1)
    def _epilogue():
        cols = []
        for hh in range(heads):
            o = acc_ref[:, hh * 128:(hh + 1) * 128]
            o = o / (den_ref[:, hh:hh + 1] + 1e-16)
            cols.append(o)
        o = jnp.concatenate(cols, axis=1) if heads > 1 else cols[0]
        o = o + b_ref[...]
        if do_elu:
            o = jnp.where(o > 0, o, jnp.exp(jnp.minimum(o, 0.0)) - 1.0)
        mu = jnp.mean(o, axis=1, keepdims=True)
        ctr = o - mu
        var = jnp.mean(ctr * ctr, axis=1, keepdims=True)
        o = ctr * lax.rsqrt(var + 1e-5) * g_ref[...] + beta_ref[...]
        if do_res:
            o = o + res_ref[...]
        out_ref[...] = o


def _gat_flash(c, xp, s_nh, d_nh, s_t, b, g, beta, res, heads, do_elu, do_res):
    dout = heads * 128
    return _pallas_call(
        functools.partial(_gat_flash_kernel, heads=heads,
                          do_elu=do_elu, do_res=do_res),
        grid=(NI, NJ),
        in_specs=[
            pl.BlockSpec((BI, BJ), lambda i, j: (i, j)),
            pl.BlockSpec((BJ, dout), lambda i, j: (j, 0)),
            pl.BlockSpec((N, heads), lambda i, j: (0, 0)),
            pl.BlockSpec((BI, heads), lambda i, j: (i, 0)),
            pl.BlockSpec((heads, BJ), lambda i, j: (0, j)),
            pl.BlockSpec((1, dout), lambda i, j: (0, 0)),
            pl.BlockSpec((1, dout), lambda i, j: (0, 0)),
            pl.BlockSpec((1, dout), lambda i, j: (0, 0)),
            pl.BlockSpec((BI, dout), lambda i, j: (i, 0)),
        ],
        out_specs=pl.BlockSpec((BI, dout), lambda i, j: (i, 0)),
        out_shape=jax.ShapeDtypeStruct((N, dout), jnp.float32),
        scratch_shapes=[
            pltpu.VMEM((BI, dout), jnp.float32),
            pltpu.VMEM((BI, heads), jnp.float32),
        ],
        compiler_params=pltpu.CompilerParams(
            dimension_semantics=("parallel", "arbitrary")),
    )(c, xp, s_nh, d_nh, s_t, b, g, beta, res)


def _gat_layer(h, c, w, asrc, adst, b, g, beta, res, heads, do_elu, do_res):
    xp, s_nh, d_nh, s_t = _gat_pre(h, w, asrc, adst, heads)
    dout = heads * 128
    b2 = b.reshape(1, dout)
    g2 = g.reshape(1, dout)
    beta2 = beta.reshape(1, dout)
    if res is None:
        res = jnp.zeros((N, dout), jnp.float32)
    return _gat_flash(c, xp, s_nh, d_nh, s_t, b2, g2, beta2, res,
                      heads, do_elu, do_res)


def _qkv_kernel(h_ref, wq_ref, bq_ref, wk_ref, bk_ref, wv_ref, bv_ref,
                q_ref, k_ref, v_ref):
    h = h_ref[...]
    q_ref[...] = _dot_nt(h, wq_ref[...]) + bq_ref[...]
    k_ref[...] = _dot_nt(h, wk_ref[...]) + bk_ref[...]
    v_ref[...] = _dot_nt(h, wv_ref[...]) + bv_ref[...]


def _qkv(h, wq, bq, wk, bk, wv, bv):
    full = lambda i: (0, 0)
    return _pallas_call(
        _qkv_kernel,
        grid=(NI,),
        in_specs=[
            pl.BlockSpec((BI, OUT), lambda i: (i, 0)),
            pl.BlockSpec((OUT, OUT), full),
            pl.BlockSpec((1, OUT), full),
            pl.BlockSpec((OUT, OUT), full),
            pl.BlockSpec((1, OUT), full),
            pl.BlockSpec((OUT, OUT), full),
            pl.BlockSpec((1, OUT), full),
        ],
        out_specs=[pl.BlockSpec((BI, OUT), lambda i: (i, 0))] * 3,
        out_shape=[jax.ShapeDtypeStruct((N, OUT), jnp.float32)] * 3,
    )(h, wq, bq.reshape(1, OUT), wk, bk.reshape(1, OUT),
      wv, bv.reshape(1, OUT))


_NEG = -1e30


def _mha_kernel(q_ref, k_ref, v_ref, wo_ref, bo_ref, out_ref,
                m_ref, l_ref, acc_ref):
    j = pl.program_id(1)

    @pl.when(j == 0)
    def _init():
        m_ref[...] = jnp.full_like(m_ref, _NEG)
        l_ref[...] = jnp.zeros_like(l_ref)
        acc_ref[...] = jnp.zeros_like(acc_ref)

    q = q_ref[...] * 0.25
    k = k_ref[...]
    v = v_ref[...]
    for hh in range(8):
        sl = slice(hh * 16, (hh + 1) * 16)
        s = _dot_nt(q[:, sl], k[:, sl])
        m_old = m_ref[:, hh:hh + 1]
        m_new = jnp.maximum(m_old, jnp.max(s, axis=1, keepdims=True))
        alpha = jnp.exp(m_old - m_new)
        p = jnp.exp(s - m_new)
        m_ref[:, hh:hh + 1] = m_new
        l_ref[:, hh:hh + 1] = l_ref[:, hh:hh + 1] * alpha + jnp.sum(
            p, axis=1, keepdims=True)
        acc_ref[:, sl] = acc_ref[:, sl] * alpha + _dot_nn(p, v[:, sl])

    @pl.when(j == NJ - 1)
    def _epilogue():
        cols = []
        for hh in range(8):
            sl = slice(hh * 16, (hh + 1) * 16)
            cols.append(acc_ref[:, sl] / l_ref[:, hh:hh + 1])
        o = jnp.concatenate(cols, axis=1)
        out_ref[...] = _dot_nt(o, wo_ref[...]) + bo_ref[...]


def _mha(q, k, v, wo, bo):
    return _pallas_call(
        _mha_kernel,
        grid=(NI, NJ),
        in_specs=[
            pl.BlockSpec((BI, OUT), lambda i, j: (i, 0)),
            pl.BlockSpec((BJ, OUT), lambda i, j: (j, 0)),
            pl.BlockSpec((BJ, OUT), lambda i, j: (j, 0)),
            pl.BlockSpec((OUT, OUT), lambda i, j: (0, 0)),
            pl.BlockSpec((1, OUT), lambda i, j: (0, 0)),
        ],
        out_specs=pl.BlockSpec((BI, OUT), lambda i, j: (i, 0)),
        out_shape=jax.ShapeDtypeStruct((N, OUT), jnp.float32),
        scratch_shapes=[
            pltpu.VMEM((BI, 8), jnp.float32),
            pltpu.VMEM((BI, 8), jnp.float32),
            pltpu.VMEM((BI, OUT), jnp.float32),
        ],
        compiler_params=pltpu.CompilerParams(
            dimension_semantics=("parallel", "arbitrary")),
    )(q, k, v, wo, bo.reshape(1, OUT))


def kernel(x, edge_index, W0, att_src0, att_dst0, b0, W1, att_src1, att_dst1,
           b1, W2, att_src2, att_dst2, b2, ln0_g, ln0_b, ln1_g, ln1_b,
           ln2_g, ln2_b, wq, bq, wk, bk, wv, bv, wo, bo):
    c = _build_count_matrix(edge_index)
    h = _gat_layer(x, c, W0, att_src0, att_dst0, b0, ln0_g, ln0_b,
                   None, heads=4, do_elu=True, do_res=False)
    h = _gat_layer(h, c, W1, att_src1, att_dst1, b1, ln1_g, ln1_b,
                   h, heads=4, do_elu=True, do_res=True)
    h = _gat_layer(h, c, W2, att_src2, att_dst2, b2, ln2_g, ln2_b,
                   None, heads=1, do_elu=False, do_res=False)
    q, k, v = _qkv(h, wq, bq, wk, bk, wv, bv)
    return _mha(q, k, v, wo, bo)
